# baseline (speedup 1.0000x reference)
import functools

import jax
import jax.numpy as jnp
from jax import lax
from jax.experimental import pallas as pl
from jax.experimental.pallas import tpu as pltpu
from jax.experimental.pallas import tpu_sc as plsc

_N = 10000
_E = 320000
_D = 128
_H = 64
_NC = 2
_NS = 16
_NW = _NC * _NS
_K = 128
_TB = 2560
_EP = _TB * _K
_NB0 = 112
_NB1 = 48
_NBMAX = max(_NB0, _NB1)
_NP = _N + 16


def _dense1_body(x_ref, wlf_ref, wlb_ref, wrf_ref, wrb_ref, blf_ref, blb_ref,
                 yf_ref, yb_ref, xr_ref):
    x = x_ref[...]
    z16 = jnp.zeros((16, _H), jnp.bfloat16)
    yf_ref[pl.ds(0, _N), :] = jnp.dot(
        x, wlf_ref[...],
        preferred_element_type=jnp.float32).astype(jnp.bfloat16)
    yf_ref[pl.ds(_N, 16), :] = z16
    yb_ref[pl.ds(0, _N), :] = jnp.dot(
        x, wlb_ref[...],
        preferred_element_type=jnp.float32).astype(jnp.bfloat16)
    yb_ref[pl.ds(_N, 16), :] = z16
    wr = wrf_ref[...] + wrb_ref[...]
    xr_ref[...] = (jnp.dot(x, wr, preferred_element_type=jnp.float32)
                   + blf_ref[...] + blb_ref[...])


def _dense1(x, wlf, wlb, wrf, wrb, blf, blb):
    f32 = jnp.float32
    return pl.pallas_call(
        _dense1_body,
        out_shape=[jax.ShapeDtypeStruct((_NP, _H), jnp.bfloat16),
                   jax.ShapeDtypeStruct((_NP, _H), jnp.bfloat16),
                   jax.ShapeDtypeStruct((_N, _H), f32)],
    )(x, wlf, wlb, wrf, wrb, blf.reshape(1, _H), blb.reshape(1, _H))


def _sc_agg_body(with_counts, yf_h, yb_h, src_h, dst_h, zrows_h, *refs):
    if with_counts:
        (zcnt_h, accf_o, accb_o, cnt_o, srcs_v, dsts_v, rows0, rows1,
         ones_v, acc_sh, cnt_sh, sem0, sem1) = refs
    else:
        (accf_o, accb_o, srcs_v, dsts_v, rows0, rows1,
         acc_sh, sem0, sem1) = refs
    cid = lax.axis_index("c")
    sid = lax.axis_index("s")
    base = jnp.where(cid == 0, sid * _NB0, _NS * _NB0 + sid * _NB1)
    nb = jnp.where(cid == 0, _NB0, _NB1)
    nbh = jnp.where(cid == 0, _NB0 // 2, _NB1 // 2)

    if with_counts:
        for j in range(_K // 16):
            ones_v[pl.ds(j * 16, 16)] = jnp.ones((16,), jnp.float32)

    pltpu.sync_copy(src_h.at[pl.ds(base, _NBMAX), :], srcs_v)
    pltpu.sync_copy(dst_h.at[pl.ds(base, _NBMAX), :], dsts_v)

    for phase in range(2):
        tbl_h = yf_h if phase == 0 else yb_h
        gat_v, sct_v = (srcs_v, dsts_v) if phase == 0 else (dsts_v, srcs_v)
        acc_o = accf_o if phase == 0 else accb_o

        @pl.when(sid < 10)
        def _zero():
            pltpu.sync_copy(zrows_h, acc_sh.at[pl.ds(sid * 1000, 1000), :])
            if with_counts:
                pltpu.sync_copy(zcnt_h, cnt_sh.at[pl.ds(sid * 1000, 1000)])
        plsc.subcore_barrier()

        pltpu.async_copy(tbl_h.at[gat_v.at[0]], rows0, sem0)

        def step(p, _):
            b = 2 * p
            pltpu.async_copy(tbl_h.at[gat_v.at[b + 1]], rows1, sem1)
            pltpu.make_async_copy(tbl_h.at[gat_v.at[b]], rows0, sem0).wait()
            pltpu.sync_copy(rows0, acc_sh.at[sct_v.at[b]], add=True)
            if with_counts:
                pltpu.sync_copy(ones_v, cnt_sh.at[sct_v.at[b]], add=True)

            @pl.when(b + 2 < nb)
            def _issue_next():
                pltpu.async_copy(tbl_h.at[gat_v.at[b + 2]], rows0, sem0)
            pltpu.make_async_copy(tbl_h.at[gat_v.at[b + 1]], rows1,
                                  sem1).wait()
            pltpu.sync_copy(rows1, acc_sh.at[sct_v.at[b + 1]], add=True)
            if with_counts:
                pltpu.sync_copy(ones_v, cnt_sh.at[sct_v.at[b + 1]],
                                add=True)
            return 0
        lax.fori_loop(0, nbh, step, 0)
        plsc.subcore_barrier()

        @pl.when(sid < 10)
        def _acc_out():
            pltpu.sync_copy(acc_sh.at[pl.ds(sid * 1000, 1000), :],
                            acc_o.at[cid, pl.ds(sid * 1000, 1000), :])
            if with_counts:
                pltpu.sync_copy(
                    cnt_sh.at[pl.ds(sid * 1000, 1000)],
                    cnt_o.at[pl.ds((cid * 2 + phase) * _N + sid * 1000,
                                   1000)])


def _sc_agg(yf, yb, src, dst, with_counts):
    f32 = jnp.float32
    mesh = plsc.VectorSubcoreMesh(core_axis_name="c", subcore_axis_name="s",
                                  num_cores=_NC, num_subcores=_NS)
    bf16 = jnp.bfloat16
    out_type = [jax.ShapeDtypeStruct((_NC, _N, _H), bf16),
                jax.ShapeDtypeStruct((_NC, _N, _H), bf16)]
    scratch = [
        pltpu.VMEM((_NBMAX, _K), jnp.int32),
        pltpu.VMEM((_NBMAX, _K), jnp.int32),
        pltpu.VMEM((_K, _H), bf16),
        pltpu.VMEM((_K, _H), bf16),
    ]
    if with_counts:
        out_type.append(jax.ShapeDtypeStruct((_NC * 2 * _N,), f32))
        scratch.append(pltpu.VMEM((_K,), f32))
    scratch.append(pltpu.VMEM_SHARED((_NP, _H), bf16))
    if with_counts:
        scratch.append(pltpu.VMEM_SHARED((_NP,), f32))
    scratch += [pltpu.SemaphoreType.DMA] * 2
    kern = pl.kernel(
        functools.partial(_sc_agg_body, with_counts),
        out_type=out_type,
        mesh=mesh,
        scratch_types=scratch,
        compiler_params=pltpu.CompilerParams(use_tc_tiling_on_sc=False),
    )
    zrows = jnp.zeros((1000, _H), bf16)
    if with_counts:
        return kern(yf, yb, src, dst, zrows, jnp.zeros((1000,), f32))
    return kern(yf, yb, src, dst, zrows)


def _combine1_body(accf_ref, accb_ref, cnt_ref, xr_ref, g_ref, be_ref,
                   wlf_ref, wlb_ref, wrf_ref, wrb_ref, blf_ref, blb_ref,
                   yf_ref, yb_ref, xr2_ref, cd_ref, cs_ref):
    sf = (accf_ref[0].astype(jnp.float32) + accf_ref[1].astype(jnp.float32))
    sb = (accb_ref[0].astype(jnp.float32) + accb_ref[1].astype(jnp.float32))
    cd = jnp.maximum(cnt_ref[0, 0] + cnt_ref[1, 0], 1.0)
    cs = jnp.maximum(cnt_ref[0, 1] + cnt_ref[1, 1], 1.0)
    cd_ref[...] = cd
    cs_ref[...] = cs
    pre = sf * (1.0 / cd)[:, None] + sb * (1.0 / cs)[:, None] + xr_ref[...]
    m = jnp.mean(pre, axis=0, keepdims=True)
    c = pre - m
    v = jnp.mean(c * c, axis=0, keepdims=True)
    h = jnp.maximum(c * jax.lax.rsqrt(v + 1e-5) * g_ref[...] + be_ref[...],
                    0.0)
    z16 = jnp.zeros((16, _H), jnp.bfloat16)
    yf_ref[pl.ds(0, _N), :] = jnp.dot(
        h, wlf_ref[...],
        preferred_element_type=jnp.float32).astype(jnp.bfloat16)
    yf_ref[pl.ds(_N, 16), :] = z16
    yb_ref[pl.ds(0, _N), :] = jnp.dot(
        h, wlb_ref[...],
        preferred_element_type=jnp.float32).astype(jnp.bfloat16)
    yb_ref[pl.ds(_N, 16), :] = z16
    wr = wrf_ref[...] + wrb_ref[...]
    xr2_ref[...] = (jnp.dot(h, wr, preferred_element_type=jnp.float32)
                    + blf_ref[...] + blb_ref[...])


def _combine1(accf, accb, cnt, xr, g, be, wlf, wlb, wrf, wrb, blf, blb):
    f32 = jnp.float32
    return pl.pallas_call(
        _combine1_body,
        out_shape=[jax.ShapeDtypeStruct((_NP, _H), jnp.bfloat16),
                   jax.ShapeDtypeStruct((_NP, _H), jnp.bfloat16),
                   jax.ShapeDtypeStruct((_N, _H), f32),
                   jax.ShapeDtypeStruct((_N,), f32),
                   jax.ShapeDtypeStruct((_N,), f32)],
    )(accf, accb, cnt, xr, g.reshape(1, _H), be.reshape(1, _H),
      wlf, wlb, wrf, wrb, blf.reshape(1, _H), blb.reshape(1, _H))


def _combine2_body(accf_ref, accb_ref, cd_ref, cs_ref, xr_ref, g_ref, be_ref,
                   out_ref):
    sf = (accf_ref[0].astype(jnp.float32) + accf_ref[1].astype(jnp.float32))
    sb = (accb_ref[0].astype(jnp.float32) + accb_ref[1].astype(jnp.float32))
    pre = (sf * (1.0 / cd_ref[...])[:, None]
           + sb * (1.0 / cs_ref[...])[:, None] + xr_ref[...])
    m = jnp.mean(pre, axis=0, keepdims=True)
    c = pre - m
    v = jnp.mean(c * c, axis=0, keepdims=True)
    h = jnp.maximum(c * jax.lax.rsqrt(v + 1e-5) * g_ref[...] + be_ref[...],
                    0.0)
    out_ref[...] = jnp.max(h, axis=0, keepdims=True)


def _combine2(accf, accb, cd, cs, xr, g, be):
    f32 = jnp.float32
    out = pl.pallas_call(
        _combine2_body,
        out_shape=jax.ShapeDtypeStruct((1, _H), f32),
    )(accf, accb, cd, cs, xr, g.reshape(1, _H), be.reshape(1, _H))
    return out.reshape(_H)


def kernel(x, edge_index, Wl_f1, bl_f1, Wr_f1, Wl_b1, bl_b1, Wr_b1,
           Wl_f2, bl_f2, Wr_f2, Wl_b2, bl_b2, Wr_b2, g1, be1, g2, be2):
    pad = jnp.full(((_TB + 64) * _K - _E,), _N, jnp.int32)
    src = jnp.concatenate([edge_index[0], pad]).reshape(_TB + 64, _K)
    dst = jnp.concatenate([edge_index[1], pad]).reshape(_TB + 64, _K)
    yf1, yb1, xr1 = _dense1(x, Wl_f1, Wl_b1, Wr_f1, Wr_b1, bl_f1, bl_b1)
    accf1, accb1, cnt_flat = _sc_agg(yf1, yb1, src, dst, with_counts=True)
    cnt = cnt_flat.reshape(_NC, 2, _N)
    yf2, yb2, xr2, cd, cs = _combine1(accf1, accb1, cnt, xr1, g1, be1,
                                      Wl_f2, Wl_b2, Wr_f2, Wr_b2,
                                      bl_f2, bl_b2)
    accf2, accb2 = _sc_agg(yf2, yb2, src, dst, with_counts=False)
    return _combine2(accf2, accb2, cd, cs, xr2, g2, be2)


# speedup vs baseline: 1.0121x; 1.0121x over previous
import functools

import jax
import jax.numpy as jnp
from jax import lax
from jax.experimental import pallas as pl
from jax.experimental.pallas import tpu as pltpu
from jax.experimental.pallas import tpu_sc as plsc

_N = 10000
_E = 320000
_D = 128
_H = 64
_NC = 2
_NS = 16
_NW = _NC * _NS
_K = 128
_TB = 2560
_EP = _TB * _K
_NB0 = 128
_NB1 = 32
_NBMAX = max(_NB0, _NB1)
_NP = _N + 16


def _dense1_body(x_ref, wlf_ref, wlb_ref, wrf_ref, wrb_ref, blf_ref, blb_ref,
                 yf_ref, yb_ref, xr_ref):
    x = x_ref[...]
    z16 = jnp.zeros((16, _H), jnp.bfloat16)
    yf_ref[pl.ds(0, _N), :] = jnp.dot(
        x, wlf_ref[...],
        preferred_element_type=jnp.float32).astype(jnp.bfloat16)
    yf_ref[pl.ds(_N, 16), :] = z16
    yb_ref[pl.ds(0, _N), :] = jnp.dot(
        x, wlb_ref[...],
        preferred_element_type=jnp.float32).astype(jnp.bfloat16)
    yb_ref[pl.ds(_N, 16), :] = z16
    wr = wrf_ref[...] + wrb_ref[...]
    xr_ref[...] = (jnp.dot(x, wr, preferred_element_type=jnp.float32)
                   + blf_ref[...] + blb_ref[...])


def _dense1(x, wlf, wlb, wrf, wrb, blf, blb):
    f32 = jnp.float32
    return pl.pallas_call(
        _dense1_body,
        out_shape=[jax.ShapeDtypeStruct((_NP, _H), jnp.bfloat16),
                   jax.ShapeDtypeStruct((_NP, _H), jnp.bfloat16),
                   jax.ShapeDtypeStruct((_N, _H), f32)],
    )(x, wlf, wlb, wrf, wrb, blf.reshape(1, _H), blb.reshape(1, _H))


def _sc_agg_body(with_counts, yf_h, yb_h, src_h, dst_h, zrows_h, *refs):
    if with_counts:
        (zcnt_h, accf_o, accb_o, cnt_o, srcs_v, dsts_v, rows0, rows1,
         ones_v, acc_sh, cnt_sh, sem0, sem1) = refs
    else:
        (accf_o, accb_o, srcs_v, dsts_v, rows0, rows1,
         acc_sh, sem0, sem1) = refs
    cid = lax.axis_index("c")
    sid = lax.axis_index("s")
    base = jnp.where(cid == 0, sid * _NB0, _NS * _NB0 + sid * _NB1)
    nb = jnp.where(cid == 0, _NB0, _NB1)
    nbh = jnp.where(cid == 0, _NB0 // 2, _NB1 // 2)

    if with_counts:
        for j in range(_K // 16):
            ones_v[pl.ds(j * 16, 16)] = jnp.ones((16,), jnp.float32)

    pltpu.sync_copy(src_h.at[pl.ds(base, _NBMAX), :], srcs_v)
    pltpu.sync_copy(dst_h.at[pl.ds(base, _NBMAX), :], dsts_v)

    for phase in range(2):
        tbl_h = yf_h if phase == 0 else yb_h
        gat_v, sct_v = (srcs_v, dsts_v) if phase == 0 else (dsts_v, srcs_v)
        acc_o = accf_o if phase == 0 else accb_o

        @pl.when(sid < 10)
        def _zero():
            pltpu.sync_copy(zrows_h, acc_sh.at[pl.ds(sid * 1000, 1000), :])
            if with_counts:
                pltpu.sync_copy(zcnt_h, cnt_sh.at[pl.ds(sid * 1000, 1000)])
        plsc.subcore_barrier()

        pltpu.async_copy(tbl_h.at[gat_v.at[0]], rows0, sem0)

        def step(p, _):
            b = 2 * p
            pltpu.async_copy(tbl_h.at[gat_v.at[b + 1]], rows1, sem1)
            pltpu.make_async_copy(tbl_h.at[gat_v.at[b]], rows0, sem0).wait()
            pltpu.sync_copy(rows0, acc_sh.at[sct_v.at[b]], add=True)
            if with_counts:
                pltpu.sync_copy(ones_v, cnt_sh.at[sct_v.at[b]], add=True)

            @pl.when(b + 2 < nb)
            def _issue_next():
                pltpu.async_copy(tbl_h.at[gat_v.at[b + 2]], rows0, sem0)
            pltpu.make_async_copy(tbl_h.at[gat_v.at[b + 1]], rows1,
                                  sem1).wait()
            pltpu.sync_copy(rows1, acc_sh.at[sct_v.at[b + 1]], add=True)
            if with_counts:
                pltpu.sync_copy(ones_v, cnt_sh.at[sct_v.at[b + 1]],
                                add=True)
            return 0
        lax.fori_loop(0, nbh, step, 0)
        plsc.subcore_barrier()

        @pl.when(sid < 10)
        def _acc_out():
            pltpu.sync_copy(acc_sh.at[pl.ds(sid * 1000, 1000), :],
                            acc_o.at[cid, pl.ds(sid * 1000, 1000), :])
            if with_counts:
                pltpu.sync_copy(
                    cnt_sh.at[pl.ds(sid * 1000, 1000)],
                    cnt_o.at[pl.ds((cid * 2 + phase) * _N + sid * 1000,
                                   1000)])


def _sc_agg(yf, yb, src, dst, with_counts):
    f32 = jnp.float32
    mesh = plsc.VectorSubcoreMesh(core_axis_name="c", subcore_axis_name="s",
                                  num_cores=_NC, num_subcores=_NS)
    bf16 = jnp.bfloat16
    out_type = [jax.ShapeDtypeStruct((_NC, _N, _H), bf16),
                jax.ShapeDtypeStruct((_NC, _N, _H), bf16)]
    scratch = [
        pltpu.VMEM((_NBMAX, _K), jnp.int32),
        pltpu.VMEM((_NBMAX, _K), jnp.int32),
        pltpu.VMEM((_K, _H), bf16),
        pltpu.VMEM((_K, _H), bf16),
    ]
    if with_counts:
        out_type.append(jax.ShapeDtypeStruct((_NC * 2 * _N,), f32))
        scratch.append(pltpu.VMEM((_K,), f32))
    scratch.append(pltpu.VMEM_SHARED((_NP, _H), bf16))
    if with_counts:
        scratch.append(pltpu.VMEM_SHARED((_NP,), f32))
    scratch += [pltpu.SemaphoreType.DMA] * 2
    kern = pl.kernel(
        functools.partial(_sc_agg_body, with_counts),
        out_type=out_type,
        mesh=mesh,
        scratch_types=scratch,
        compiler_params=pltpu.CompilerParams(use_tc_tiling_on_sc=False),
    )
    zrows = jnp.zeros((1000, _H), bf16)
    if with_counts:
        return kern(yf, yb, src, dst, zrows, jnp.zeros((1000,), f32))
    return kern(yf, yb, src, dst, zrows)


def _combine1_body(accf_ref, accb_ref, cnt_ref, xr_ref, g_ref, be_ref,
                   wlf_ref, wlb_ref, wrf_ref, wrb_ref, blf_ref, blb_ref,
                   yf_ref, yb_ref, xr2_ref, cd_ref, cs_ref):
    sf = (accf_ref[0].astype(jnp.float32) + accf_ref[1].astype(jnp.float32))
    sb = (accb_ref[0].astype(jnp.float32) + accb_ref[1].astype(jnp.float32))
    cd = jnp.maximum(cnt_ref[0, 0] + cnt_ref[1, 0], 1.0)
    cs = jnp.maximum(cnt_ref[0, 1] + cnt_ref[1, 1], 1.0)
    cd_ref[...] = cd
    cs_ref[...] = cs
    pre = sf * (1.0 / cd)[:, None] + sb * (1.0 / cs)[:, None] + xr_ref[...]
    m = jnp.mean(pre, axis=0, keepdims=True)
    c = pre - m
    v = jnp.mean(c * c, axis=0, keepdims=True)
    h = jnp.maximum(c * jax.lax.rsqrt(v + 1e-5) * g_ref[...] + be_ref[...],
                    0.0)
    z16 = jnp.zeros((16, _H), jnp.bfloat16)
    yf_ref[pl.ds(0, _N), :] = jnp.dot(
        h, wlf_ref[...],
        preferred_element_type=jnp.float32).astype(jnp.bfloat16)
    yf_ref[pl.ds(_N, 16), :] = z16
    yb_ref[pl.ds(0, _N), :] = jnp.dot(
        h, wlb_ref[...],
        preferred_element_type=jnp.float32).astype(jnp.bfloat16)
    yb_ref[pl.ds(_N, 16), :] = z16
    wr = wrf_ref[...] + wrb_ref[...]
    xr2_ref[...] = (jnp.dot(h, wr, preferred_element_type=jnp.float32)
                    + blf_ref[...] + blb_ref[...])


def _combine1(accf, accb, cnt, xr, g, be, wlf, wlb, wrf, wrb, blf, blb):
    f32 = jnp.float32
    return pl.pallas_call(
        _combine1_body,
        out_shape=[jax.ShapeDtypeStruct((_NP, _H), jnp.bfloat16),
                   jax.ShapeDtypeStruct((_NP, _H), jnp.bfloat16),
                   jax.ShapeDtypeStruct((_N, _H), f32),
                   jax.ShapeDtypeStruct((_N,), f32),
                   jax.ShapeDtypeStruct((_N,), f32)],
    )(accf, accb, cnt, xr, g.reshape(1, _H), be.reshape(1, _H),
      wlf, wlb, wrf, wrb, blf.reshape(1, _H), blb.reshape(1, _H))


def _combine2_body(accf_ref, accb_ref, cd_ref, cs_ref, xr_ref, g_ref, be_ref,
                   out_ref):
    sf = (accf_ref[0].astype(jnp.float32) + accf_ref[1].astype(jnp.float32))
    sb = (accb_ref[0].astype(jnp.float32) + accb_ref[1].astype(jnp.float32))
    pre = (sf * (1.0 / cd_ref[...])[:, None]
           + sb * (1.0 / cs_ref[...])[:, None] + xr_ref[...])
    m = jnp.mean(pre, axis=0, keepdims=True)
    c = pre - m
    v = jnp.mean(c * c, axis=0, keepdims=True)
    h = jnp.maximum(c * jax.lax.rsqrt(v + 1e-5) * g_ref[...] + be_ref[...],
                    0.0)
    out_ref[...] = jnp.max(h, axis=0, keepdims=True)


def _combine2(accf, accb, cd, cs, xr, g, be):
    f32 = jnp.float32
    out = pl.pallas_call(
        _combine2_body,
        out_shape=jax.ShapeDtypeStruct((1, _H), f32),
    )(accf, accb, cd, cs, xr, g.reshape(1, _H), be.reshape(1, _H))
    return out.reshape(_H)


def kernel(x, edge_index, Wl_f1, bl_f1, Wr_f1, Wl_b1, bl_b1, Wr_b1,
           Wl_f2, bl_f2, Wr_f2, Wl_b2, bl_b2, Wr_b2, g1, be1, g2, be2):
    pad = jnp.full(((_TB + 64) * _K - _E,), _N, jnp.int32)
    src = jnp.concatenate([edge_index[0], pad]).reshape(_TB + 64, _K)
    dst = jnp.concatenate([edge_index[1], pad]).reshape(_TB + 64, _K)
    yf1, yb1, xr1 = _dense1(x, Wl_f1, Wl_b1, Wr_f1, Wr_b1, bl_f1, bl_b1)
    accf1, accb1, cnt_flat = _sc_agg(yf1, yb1, src, dst, with_counts=True)
    cnt = cnt_flat.reshape(_NC, 2, _N)
    yf2, yb2, xr2, cd, cs = _combine1(accf1, accb1, cnt, xr1, g1, be1,
                                      Wl_f2, Wl_b2, Wr_f2, Wr_b2,
                                      bl_f2, bl_b2)
    accf2, accb2 = _sc_agg(yf2, yb2, src, dst, with_counts=False)
    return _combine2(accf2, accb2, cd, cs, xr2, g2, be2)
